# Initial kernel scaffold; baseline (speedup 1.0000x reference)
import jax
import jax.numpy as jnp
from jax.experimental import pallas as pl


def kernel(x, edge_index, W1, b1, W2, b2):
    raise NotImplementedError("write your pallas kernel here")


# speedup vs baseline: 155.6648x; 155.6648x over previous
import functools

import jax
import jax.numpy as jnp
from jax import lax
from jax.experimental import pallas as pl
from jax.experimental.pallas import tpu as pltpu
from jax.experimental.pallas import tpu_sc as plsc

N_NODES = 100000
N_EDGES = 3200000
NC, NS, L = 2, 16, 16
NW = NC * NS
CH = 128
BLK_ROWS = 8
BLK = BLK_ROWS * CH
NBLK = N_EDGES // BLK
KMAX = -(-NBLK // NW)
NPAD = 102400
ROWS128 = NPAD // 128
TSLICE = NPAD // NS


def _sc_pass_body(ei_ref, g_ref, zero_ref, out_ref, gtab, rows_v, cols_v,
                  vals_v, bounce, acc, sem):
    c = lax.axis_index("c")
    s = lax.axis_index("s")
    wid = s * NC + c

    pltpu.sync_copy(g_ref, gtab)
    pltpu.sync_copy(zero_ref.at[pl.ds(s * TSLICE, TSLICE)],
                    acc.at[pl.ds(s * TSLICE, TSLICE)])
    plsc.subcore_barrier()

    def blk(k, carry):
        b = wid + NW * k

        @pl.when(b < NBLK)
        def _():
            pltpu.sync_copy(ei_ref.at[0, pl.ds(b * BLK_ROWS, BLK_ROWS)], rows_v)
            pltpu.sync_copy(ei_ref.at[1, pl.ds(b * BLK_ROWS, BLK_ROWS)], cols_v)
            for j in range(BLK_ROWS):
                for i in range(CH // L):
                    idx = rows_v[j, pl.ds(i * L, L)]
                    vals_v[j, pl.ds(i * L, L)] = plsc.load_gather(gtab, [idx])
            copies = [
                pltpu.async_copy(vals_v.at[j], acc.at[cols_v.at[j]], sem,
                                 add=True)
                for j in range(BLK_ROWS)
            ]
            for cp in copies:
                cp.wait()

        return carry

    lax.fori_loop(0, KMAX, blk, 0)
    plsc.subcore_barrier()

    pltpu.sync_copy(acc.at[pl.ds(s * TSLICE, TSLICE)], bounce)
    pltpu.sync_copy(bounce, out_ref.at[c, pl.ds(s * TSLICE, TSLICE)])


_sc_pass = pl.kernel(
    _sc_pass_body,
    out_type=jax.ShapeDtypeStruct((NC, NPAD), jnp.float32),
    mesh=plsc.VectorSubcoreMesh(core_axis_name="c", subcore_axis_name="s",
                                num_cores=NC, num_subcores=NS),
    scratch_types=[
        pltpu.VMEM((NPAD,), jnp.float32),
        pltpu.VMEM((BLK_ROWS, CH), jnp.int32),
        pltpu.VMEM((BLK_ROWS, CH), jnp.int32),
        pltpu.VMEM((BLK_ROWS, CH), jnp.float32),
        pltpu.VMEM((TSLICE,), jnp.float32),
        pltpu.VMEM_SHARED((NPAD,), jnp.float32),
        pltpu.SemaphoreType.DMA,
    ],
    compiler_params=pltpu.CompilerParams(needs_layout_passes=False),
)


def _ew1_body(d_ref, x_ref, dis_ref, g_ref):
    deg = d_ref[0] + d_ref[1] + 1.0
    dis = lax.rsqrt(deg)
    dis_ref[...] = dis
    g_ref[...] = dis * x_ref[...]


def _ew2_body(p_ref, g_ref, dis_ref, w1_ref, b1_ref, w2_ref, g2_ref):
    dis = dis_ref[...]
    sv = dis * (p_ref[0] + p_ref[1] + g_ref[...])
    t = jnp.zeros_like(sv)
    for k in range(16):
        t = t + jnp.maximum(sv * w1_ref[0, k] + b1_ref[k], 0.0) * w2_ref[k, 0]
    g2_ref[...] = dis * t


def _ew3_body(p_ref, g2_ref, dis_ref, b2_ref, o_ref):
    u = dis_ref[...] * (p_ref[0] + p_ref[1] + g2_ref[...]) + b2_ref[0]
    o_ref[...] = 1.0 / (1.0 + jnp.exp(-u))


_V = functools.partial(pl.BlockSpec, memory_space=pltpu.MemorySpace.VMEM)
_S = functools.partial(pl.BlockSpec, memory_space=pltpu.MemorySpace.SMEM)
_F = jax.ShapeDtypeStruct((ROWS128, 128), jnp.float32)

_ew1 = pl.pallas_call(_ew1_body, out_shape=(_F, _F),
                      in_specs=[_V(), _V()], out_specs=(_V(), _V()))
_ew2 = pl.pallas_call(_ew2_body, out_shape=_F,
                      in_specs=[_V(), _V(), _V(), _S(), _S(), _S()],
                      out_specs=_V())
_ew3 = pl.pallas_call(_ew3_body, out_shape=_F,
                      in_specs=[_V(), _V(), _V(), _S()], out_specs=_V())


def kernel(x, edge_index, W1, b1, W2, b2):
    ei = edge_index.astype(jnp.int32).reshape(2, N_EDGES // CH, CH)
    xp = jnp.pad(x.astype(jnp.float32), (0, NPAD - N_NODES))
    zeros = jnp.zeros((NPAD,), jnp.float32)
    ones = jnp.ones((NPAD,), jnp.float32)

    deg_p = _sc_pass(ei, ones, zeros)
    dis, g = _ew1(deg_p.reshape(NC, ROWS128, 128), xp.reshape(ROWS128, 128))

    s_p = _sc_pass(ei, g.reshape(NPAD), zeros)
    g2 = _ew2(s_p.reshape(NC, ROWS128, 128), g, dis, W1, b1, W2)

    u_p = _sc_pass(ei, g2.reshape(NPAD), zeros)
    out = _ew3(u_p.reshape(NC, ROWS128, 128), g2, dis, b2)

    return out.reshape(NPAD)[:N_NODES].reshape(N_NODES, 1)


# speedup vs baseline: 343.8139x; 2.2087x over previous
import functools

import jax
import jax.numpy as jnp
from jax import lax
from jax.experimental import pallas as pl
from jax.experimental.pallas import tpu as pltpu
from jax.experimental.pallas import tpu_sc as plsc

N_NODES = 100000
N_EDGES = 3200000
NC, NS, L = 2, 16, 16
NW = NC * NS
CH = 128
BLK_ROWS = 8
BLK = BLK_ROWS * CH
NBLK = N_EDGES // BLK
KMAX = -(-NBLK // NW)
PAIRS = (KMAX + 1) // 2
NPAD = 102400
ROWS128 = NPAD // 128
TSLICE = NPAD // NS

_SC_PARAMS = pltpu.CompilerParams(needs_layout_passes=False)
_MESH = plsc.VectorSubcoreMesh(core_axis_name="c", subcore_axis_name="s",
                               num_cores=NC, num_subcores=NS)


def _stage(ei_ref, dim, b, dst, sem):
    return pltpu.async_copy(
        ei_ref.at[dim, pl.ds(b * BLK_ROWS, BLK_ROWS)], dst, sem)


def _gather_pass_body(ei_ref, g_ref, zero_ref, out_ref, gtab,
                      rows_a, cols_a, vals_a, rows_b, cols_b, vals_b,
                      bounce, acc, sem_a, sem_b, sem_sa, sem_sb):
    c = lax.axis_index("c")
    s = lax.axis_index("s")
    wid = s * NC + c

    pltpu.sync_copy(g_ref, gtab)
    pltpu.sync_copy(zero_ref.at[pl.ds(s * TSLICE, TSLICE)],
                    acc.at[pl.ds(s * TSLICE, TSLICE)])
    plsc.subcore_barrier()

    _stage(ei_ref, 0, wid, rows_a, sem_a)
    _stage(ei_ref, 1, wid, cols_a, sem_a)
    _stage(ei_ref, 0, wid + NW, rows_b, sem_b)
    _stage(ei_ref, 1, wid + NW, cols_b, sem_b)

    def run_block(b, rows_v, cols_v, vals_v, sem, sem_s):
        pltpu.make_async_copy(
            ei_ref.at[0, pl.ds(b * BLK_ROWS, BLK_ROWS)], rows_v, sem).wait()
        pltpu.make_async_copy(
            ei_ref.at[1, pl.ds(b * BLK_ROWS, BLK_ROWS)], cols_v, sem).wait()
        for j in range(BLK_ROWS):
            for i in range(CH // L):
                idx = rows_v[j, pl.ds(i * L, L)]
                vals_v[j, pl.ds(i * L, L)] = plsc.load_gather(gtab, [idx])
        return [
            pltpu.async_copy(vals_v.at[j], acc.at[cols_v.at[j]], sem_s,
                             add=True)
            for j in range(BLK_ROWS)
        ]

    def pair(k2, carry):
        b0 = wid + NW * (2 * k2)
        b1 = b0 + NW
        scat_a = []
        scat_b = []

        @pl.when(b0 < NBLK)
        def _():
            cps = run_block(b0, rows_a, cols_a, vals_a, sem_a, sem_sa)
            scat_a.extend(cps)

        @pl.when(b1 < NBLK)
        def _():
            cps = run_block(b1, rows_b, cols_b, vals_b, sem_b, sem_sb)
            scat_b.extend(cps)

        @pl.when(b0 < NBLK)
        def _():
            for cp in scat_a:
                cp.wait()

        @pl.when(b0 + 2 * NW < NBLK)
        def _():
            _stage(ei_ref, 0, b0 + 2 * NW, rows_a, sem_a)
            _stage(ei_ref, 1, b0 + 2 * NW, cols_a, sem_a)

        @pl.when(b1 < NBLK)
        def _():
            for cp in scat_b:
                cp.wait()

        @pl.when(b1 + 2 * NW < NBLK)
        def _():
            _stage(ei_ref, 0, b1 + 2 * NW, rows_b, sem_b)
            _stage(ei_ref, 1, b1 + 2 * NW, cols_b, sem_b)

        return carry

    lax.fori_loop(0, PAIRS, pair, 0)
    plsc.subcore_barrier()

    pltpu.sync_copy(acc.at[pl.ds(s * TSLICE, TSLICE)], bounce)
    pltpu.sync_copy(bounce, out_ref.at[c, pl.ds(s * TSLICE, TSLICE)])


_gather_pass = pl.kernel(
    _gather_pass_body,
    out_type=jax.ShapeDtypeStruct((NC, NPAD), jnp.float32),
    mesh=_MESH,
    scratch_types=[
        pltpu.VMEM((NPAD,), jnp.float32),
        pltpu.VMEM((BLK_ROWS, CH), jnp.int32),
        pltpu.VMEM((BLK_ROWS, CH), jnp.int32),
        pltpu.VMEM((BLK_ROWS, CH), jnp.float32),
        pltpu.VMEM((BLK_ROWS, CH), jnp.int32),
        pltpu.VMEM((BLK_ROWS, CH), jnp.int32),
        pltpu.VMEM((BLK_ROWS, CH), jnp.float32),
        pltpu.VMEM((TSLICE,), jnp.float32),
        pltpu.VMEM_SHARED((NPAD,), jnp.float32),
        pltpu.SemaphoreType.DMA,
        pltpu.SemaphoreType.DMA,
        pltpu.SemaphoreType.DMA,
        pltpu.SemaphoreType.DMA,
    ],
    compiler_params=_SC_PARAMS,
)


def _deg_pass_body(ei_ref, zero_ref, out_ref, cols_a, cols_b, ones_v,
                   bounce, acc, sem_a, sem_b, sem_sa, sem_sb):
    c = lax.axis_index("c")
    s = lax.axis_index("s")
    wid = s * NC + c

    for i in range(CH // L):
        ones_v[pl.ds(i * L, L)] = jnp.ones((L,), jnp.float32)
    pltpu.sync_copy(zero_ref.at[pl.ds(s * TSLICE, TSLICE)],
                    acc.at[pl.ds(s * TSLICE, TSLICE)])
    plsc.subcore_barrier()

    _stage(ei_ref, 1, wid, cols_a, sem_a)
    _stage(ei_ref, 1, wid + NW, cols_b, sem_b)

    def run_block(b, cols_v, sem, sem_s):
        pltpu.make_async_copy(
            ei_ref.at[1, pl.ds(b * BLK_ROWS, BLK_ROWS)], cols_v, sem).wait()
        return [
            pltpu.async_copy(ones_v, acc.at[cols_v.at[j]], sem_s, add=True)
            for j in range(BLK_ROWS)
        ]

    def pair(k2, carry):
        b0 = wid + NW * (2 * k2)
        b1 = b0 + NW
        scat_a = []
        scat_b = []

        @pl.when(b0 < NBLK)
        def _():
            scat_a.extend(run_block(b0, cols_a, sem_a, sem_sa))

        @pl.when(b1 < NBLK)
        def _():
            scat_b.extend(run_block(b1, cols_b, sem_b, sem_sb))

        @pl.when(b0 < NBLK)
        def _():
            for cp in scat_a:
                cp.wait()

        @pl.when(b0 + 2 * NW < NBLK)
        def _():
            _stage(ei_ref, 1, b0 + 2 * NW, cols_a, sem_a)

        @pl.when(b1 < NBLK)
        def _():
            for cp in scat_b:
                cp.wait()

        @pl.when(b1 + 2 * NW < NBLK)
        def _():
            _stage(ei_ref, 1, b1 + 2 * NW, cols_b, sem_b)

        return carry

    lax.fori_loop(0, PAIRS, pair, 0)
    plsc.subcore_barrier()

    pltpu.sync_copy(acc.at[pl.ds(s * TSLICE, TSLICE)], bounce)
    pltpu.sync_copy(bounce, out_ref.at[c, pl.ds(s * TSLICE, TSLICE)])


_deg_pass = pl.kernel(
    _deg_pass_body,
    out_type=jax.ShapeDtypeStruct((NC, NPAD), jnp.float32),
    mesh=_MESH,
    scratch_types=[
        pltpu.VMEM((BLK_ROWS, CH), jnp.int32),
        pltpu.VMEM((BLK_ROWS, CH), jnp.int32),
        pltpu.VMEM((CH,), jnp.float32),
        pltpu.VMEM((TSLICE,), jnp.float32),
        pltpu.VMEM_SHARED((NPAD,), jnp.float32),
        pltpu.SemaphoreType.DMA,
        pltpu.SemaphoreType.DMA,
        pltpu.SemaphoreType.DMA,
        pltpu.SemaphoreType.DMA,
    ],
    compiler_params=_SC_PARAMS,
)


def _ew1_body(d_ref, x_ref, dis_ref, g_ref):
    deg = d_ref[0] + d_ref[1] + 1.0
    dis = lax.rsqrt(deg)
    dis_ref[...] = dis
    g_ref[...] = dis * x_ref[...]


def _ew2_body(p_ref, g_ref, dis_ref, w1_ref, b1_ref, w2_ref, g2_ref):
    dis = dis_ref[...]
    sv = dis * (p_ref[0] + p_ref[1] + g_ref[...])
    t = jnp.zeros_like(sv)
    for k in range(16):
        t = t + jnp.maximum(sv * w1_ref[0, k] + b1_ref[k], 0.0) * w2_ref[k, 0]
    g2_ref[...] = dis * t


def _ew3_body(p_ref, g2_ref, dis_ref, b2_ref, o_ref):
    u = dis_ref[...] * (p_ref[0] + p_ref[1] + g2_ref[...]) + b2_ref[0]
    o_ref[...] = 1.0 / (1.0 + jnp.exp(-u))


_V = functools.partial(pl.BlockSpec, memory_space=pltpu.MemorySpace.VMEM)
_S = functools.partial(pl.BlockSpec, memory_space=pltpu.MemorySpace.SMEM)
_F = jax.ShapeDtypeStruct((ROWS128, 128), jnp.float32)

_ew1 = pl.pallas_call(_ew1_body, out_shape=(_F, _F),
                      in_specs=[_V(), _V()], out_specs=(_V(), _V()))
_ew2 = pl.pallas_call(_ew2_body, out_shape=_F,
                      in_specs=[_V(), _V(), _V(), _S(), _S(), _S()],
                      out_specs=_V())
_ew3 = pl.pallas_call(_ew3_body, out_shape=_F,
                      in_specs=[_V(), _V(), _V(), _S()], out_specs=_V())


def kernel(x, edge_index, W1, b1, W2, b2):
    ei = edge_index.astype(jnp.int32).reshape(2, N_EDGES // CH, CH)
    xp = jnp.pad(x.astype(jnp.float32), (0, NPAD - N_NODES))
    zeros = jnp.zeros((NPAD,), jnp.float32)

    deg_p = _deg_pass(ei, zeros)
    dis, g = _ew1(deg_p.reshape(NC, ROWS128, 128), xp.reshape(ROWS128, 128))

    s_p = _gather_pass(ei, g.reshape(NPAD), zeros)
    g2 = _ew2(s_p.reshape(NC, ROWS128, 128), g, dis, W1, b1, W2)

    u_p = _gather_pass(ei, g2.reshape(NPAD), zeros)
    out = _ew3(u_p.reshape(NC, ROWS128, 128), g2, dis, b2)

    return out.reshape(NPAD)[:N_NODES].reshape(N_NODES, 1)


# speedup vs baseline: 378.7101x; 1.1015x over previous
import functools

import jax
import jax.numpy as jnp
from jax import lax
from jax.experimental import pallas as pl
from jax.experimental.pallas import tpu as pltpu
from jax.experimental.pallas import tpu_sc as plsc

N_NODES = 100000
N_EDGES = 3200000
NC, NS, L = 2, 16, 16
NW = NC * NS
CH = 128
BLK_ROWS = 8
BLK = BLK_ROWS * CH
NBLK = N_EDGES // BLK
KMAX = -(-NBLK // NW)
PAIRS = (KMAX + 1) // 2
NPAD = 102400
ROWS128 = NPAD // 128
TSLICE = NPAD // NS

_SC_PARAMS = pltpu.CompilerParams(needs_layout_passes=False)
_MESH = plsc.VectorSubcoreMesh(core_axis_name="c", subcore_axis_name="s",
                               num_cores=NC, num_subcores=NS)


def _stage(ei_ref, b, dst, sem):
    return pltpu.async_copy(
        ei_ref.at[pl.ds(b * 2 * BLK_ROWS, 2 * BLK_ROWS)], dst, sem)


def _gather_pass_body(ei_ref, g_ref, zero_ref, out_ref, gtab,
                      ev_a, vals_a, ev_b, vals_b,
                      bounce, acc, sem_a, sem_b, sem_sa, sem_sb):
    c = lax.axis_index("c")
    s = lax.axis_index("s")
    wid = s * NC + c

    pltpu.sync_copy(g_ref, gtab)
    pltpu.sync_copy(zero_ref.at[pl.ds(s * TSLICE, TSLICE)],
                    acc.at[pl.ds(s * TSLICE, TSLICE)])
    plsc.subcore_barrier()

    _stage(ei_ref, wid, ev_a, sem_a)
    _stage(ei_ref, wid + NW, ev_b, sem_b)

    def run_block(b, ev_v, vals_v, sem, sem_s):
        pltpu.make_async_copy(
            ei_ref.at[pl.ds(b * 2 * BLK_ROWS, 2 * BLK_ROWS)], ev_v, sem).wait()
        for j in range(BLK_ROWS):
            for i in range(CH // L):
                idx = ev_v[2 * j, pl.ds(i * L, L)]
                vals_v[j, pl.ds(i * L, L)] = plsc.load_gather(gtab, [idx])
        return [
            pltpu.async_copy(vals_v.at[j], acc.at[ev_v.at[2 * j + 1]], sem_s,
                             add=True)
            for j in range(BLK_ROWS)
        ]

    def pair(k2, carry):
        b0 = wid + NW * (2 * k2)
        b1 = b0 + NW
        scat_a = []
        scat_b = []

        @pl.when(b0 < NBLK)
        def _():
            scat_a.extend(run_block(b0, ev_a, vals_a, sem_a, sem_sa))

        @pl.when(b1 < NBLK)
        def _():
            scat_b.extend(run_block(b1, ev_b, vals_b, sem_b, sem_sb))

        @pl.when(b0 < NBLK)
        def _():
            for cp in scat_a:
                cp.wait()

        @pl.when(b0 + 2 * NW < NBLK)
        def _():
            _stage(ei_ref, b0 + 2 * NW, ev_a, sem_a)

        @pl.when(b1 < NBLK)
        def _():
            for cp in scat_b:
                cp.wait()

        @pl.when(b1 + 2 * NW < NBLK)
        def _():
            _stage(ei_ref, b1 + 2 * NW, ev_b, sem_b)

        return carry

    lax.fori_loop(0, PAIRS, pair, 0)
    plsc.subcore_barrier()

    pltpu.sync_copy(acc.at[pl.ds(s * TSLICE, TSLICE)], bounce)
    pltpu.sync_copy(bounce, out_ref.at[pl.ds(c * NPAD + s * TSLICE, TSLICE)])


_gather_pass = pl.kernel(
    _gather_pass_body,
    out_type=jax.ShapeDtypeStruct((NC * NPAD,), jnp.float32),
    mesh=_MESH,
    scratch_types=[
        pltpu.VMEM((NPAD,), jnp.float32),
        pltpu.VMEM((2 * BLK_ROWS, CH), jnp.int32),
        pltpu.VMEM((BLK_ROWS, CH), jnp.float32),
        pltpu.VMEM((2 * BLK_ROWS, CH), jnp.int32),
        pltpu.VMEM((BLK_ROWS, CH), jnp.float32),
        pltpu.VMEM((TSLICE,), jnp.float32),
        pltpu.VMEM_SHARED((NPAD,), jnp.float32),
        pltpu.SemaphoreType.DMA,
        pltpu.SemaphoreType.DMA,
        pltpu.SemaphoreType.DMA,
        pltpu.SemaphoreType.DMA,
    ],
    compiler_params=_SC_PARAMS,
)


def _deg_pass_body(ei_ref, zero_ref, out_ref, ev_a, ev_b, ones_v,
                   bounce, acc, sem_a, sem_b, sem_sa, sem_sb):
    c = lax.axis_index("c")
    s = lax.axis_index("s")
    wid = s * NC + c

    for i in range(CH // L):
        ones_v[pl.ds(i * L, L)] = jnp.ones((L,), jnp.float32)
    pltpu.sync_copy(zero_ref.at[pl.ds(s * TSLICE, TSLICE)],
                    acc.at[pl.ds(s * TSLICE, TSLICE)])
    plsc.subcore_barrier()

    _stage(ei_ref, wid, ev_a, sem_a)
    _stage(ei_ref, wid + NW, ev_b, sem_b)

    def run_block(b, ev_v, sem, sem_s):
        pltpu.make_async_copy(
            ei_ref.at[pl.ds(b * 2 * BLK_ROWS, 2 * BLK_ROWS)], ev_v, sem).wait()
        return [
            pltpu.async_copy(ones_v, acc.at[ev_v.at[2 * j + 1]], sem_s,
                             add=True)
            for j in range(BLK_ROWS)
        ]

    def pair(k2, carry):
        b0 = wid + NW * (2 * k2)
        b1 = b0 + NW
        scat_a = []
        scat_b = []

        @pl.when(b0 < NBLK)
        def _():
            scat_a.extend(run_block(b0, ev_a, sem_a, sem_sa))

        @pl.when(b1 < NBLK)
        def _():
            scat_b.extend(run_block(b1, ev_b, sem_b, sem_sb))

        @pl.when(b0 < NBLK)
        def _():
            for cp in scat_a:
                cp.wait()

        @pl.when(b0 + 2 * NW < NBLK)
        def _():
            _stage(ei_ref, b0 + 2 * NW, ev_a, sem_a)

        @pl.when(b1 < NBLK)
        def _():
            for cp in scat_b:
                cp.wait()

        @pl.when(b1 + 2 * NW < NBLK)
        def _():
            _stage(ei_ref, b1 + 2 * NW, ev_b, sem_b)

        return carry

    lax.fori_loop(0, PAIRS, pair, 0)
    plsc.subcore_barrier()

    pltpu.sync_copy(acc.at[pl.ds(s * TSLICE, TSLICE)], bounce)
    pltpu.sync_copy(bounce, out_ref.at[pl.ds(c * NPAD + s * TSLICE, TSLICE)])


_deg_pass = pl.kernel(
    _deg_pass_body,
    out_type=jax.ShapeDtypeStruct((NC * NPAD,), jnp.float32),
    mesh=_MESH,
    scratch_types=[
        pltpu.VMEM((2 * BLK_ROWS, CH), jnp.int32),
        pltpu.VMEM((2 * BLK_ROWS, CH), jnp.int32),
        pltpu.VMEM((CH,), jnp.float32),
        pltpu.VMEM((TSLICE,), jnp.float32),
        pltpu.VMEM_SHARED((NPAD,), jnp.float32),
        pltpu.SemaphoreType.DMA,
        pltpu.SemaphoreType.DMA,
        pltpu.SemaphoreType.DMA,
        pltpu.SemaphoreType.DMA,
    ],
    compiler_params=_SC_PARAMS,
)


def _ew1_body(d_ref, x_ref, dis_ref, g_ref):
    deg = d_ref[0] + d_ref[1] + 1.0
    dis = lax.rsqrt(deg)
    dis_ref[...] = dis
    g_ref[...] = dis * x_ref[...]


def _ew2_body(p_ref, g_ref, dis_ref, w1_ref, b1_ref, w2_ref, g2_ref):
    dis = dis_ref[...]
    sv = dis * (p_ref[0] + p_ref[1] + g_ref[...])
    t = jnp.zeros_like(sv)
    for k in range(16):
        t = t + jnp.maximum(sv * w1_ref[0, k] + b1_ref[k], 0.0) * w2_ref[k, 0]
    g2_ref[...] = dis * t


def _ew3_body(p_ref, g2_ref, dis_ref, b2_ref, o_ref):
    u = dis_ref[...] * (p_ref[0] + p_ref[1] + g2_ref[...]) + b2_ref[0]
    o_ref[...] = 1.0 / (1.0 + jnp.exp(-u))


_V = functools.partial(pl.BlockSpec, memory_space=pltpu.MemorySpace.VMEM)
_S = functools.partial(pl.BlockSpec, memory_space=pltpu.MemorySpace.SMEM)
_F = jax.ShapeDtypeStruct((ROWS128, 128), jnp.float32)

_ew1 = pl.pallas_call(_ew1_body, out_shape=(_F, _F),
                      in_specs=[_V(), _V()], out_specs=(_V(), _V()))
_ew2 = pl.pallas_call(_ew2_body, out_shape=_F,
                      in_specs=[_V(), _V(), _V(), _S(), _S(), _S()],
                      out_specs=_V())
_ew3 = pl.pallas_call(_ew3_body, out_shape=_F,
                      in_specs=[_V(), _V(), _V(), _S()], out_specs=_V())


def kernel(x, edge_index, W1, b1, W2, b2):
    ei = (edge_index.astype(jnp.int32)
          .reshape(2, N_EDGES // CH, CH)
          .transpose(1, 0, 2)
          .reshape(2 * (N_EDGES // CH), CH))
    xp = jnp.pad(x.astype(jnp.float32), (0, NPAD - N_NODES))
    zeros = jnp.zeros((NPAD,), jnp.float32)

    deg_p = _deg_pass(ei, zeros)
    dis, g = _ew1(deg_p.reshape(NC, ROWS128, 128), xp.reshape(ROWS128, 128))

    s_p = _gather_pass(ei, g.reshape(NPAD), zeros)
    g2 = _ew2(s_p.reshape(NC, ROWS128, 128), g, dis, W1, b1, W2)

    u_p = _gather_pass(ei, g2.reshape(NPAD), zeros)
    out = _ew3(u_p.reshape(NC, ROWS128, 128), g2, dis, b2)

    return out.reshape(NPAD)[:N_NODES].reshape(N_NODES, 1)


# speedup vs baseline: 393.2326x; 1.0383x over previous
import functools

import jax
import jax.numpy as jnp
from jax import lax
from jax.experimental import pallas as pl
from jax.experimental.pallas import tpu as pltpu
from jax.experimental.pallas import tpu_sc as plsc

N_NODES = 100000
N_EDGES = 3200000
NC, NS, L = 2, 16, 16
NW = NC * NS
CH = 128
BLK_ROWS = 8
BLK = BLK_ROWS * CH
NBLK = N_EDGES // BLK
KMAX = -(-NBLK // NW)
PAIRS = (KMAX + 1) // 2
NPAD = 102400
ROWS128 = NPAD // 128
TSLICE = NPAD // NS

_SC_PARAMS = pltpu.CompilerParams(needs_layout_passes=False)
_MESH = plsc.VectorSubcoreMesh(core_axis_name="c", subcore_axis_name="s",
                               num_cores=NC, num_subcores=NS)


def _stage(ei_ref, b, dst, sem):
    return pltpu.async_copy(
        ei_ref.at[pl.ds(b * 2 * BLK_ROWS, 2 * BLK_ROWS)], dst, sem)


def _gather_pass_body(ei_ref, g_ref, zero_ref, out_ref, gtab,
                      ev_a, cols_a, vals_a, ev_b, cols_b, vals_b,
                      bounce, acc, sem_a, sem_b, sem_sa, sem_sb):
    c = lax.axis_index("c")
    s = lax.axis_index("s")
    wid = s * NC + c

    pltpu.sync_copy(g_ref, gtab)
    pltpu.sync_copy(zero_ref.at[pl.ds(s * TSLICE, TSLICE)],
                    acc.at[pl.ds(s * TSLICE, TSLICE)])
    plsc.subcore_barrier()

    _stage(ei_ref, wid, ev_a, sem_a)
    _stage(ei_ref, wid + NW, ev_b, sem_b)

    def run_block(b, ev_v, cols_v, vals_v, sem, sem_s):
        pltpu.make_async_copy(
            ei_ref.at[pl.ds(b * 2 * BLK_ROWS, 2 * BLK_ROWS)], ev_v, sem).wait()
        for j in range(BLK_ROWS):
            for i in range(CH // L):
                idx = ev_v[2 * j, pl.ds(i * L, L)]
                vals_v[pl.ds(j * CH + i * L, L)] = plsc.load_gather(gtab, [idx])
                cols_v[pl.ds(j * CH + i * L, L)] = (
                    ev_v[2 * j + 1, pl.ds(i * L, L)])
        return [pltpu.async_copy(vals_v, acc.at[cols_v], sem_s, add=True)]

    def pair(k2, carry):
        b0 = wid + NW * (2 * k2)
        b1 = b0 + NW
        scat_a = []
        scat_b = []

        @pl.when(b0 < NBLK)
        def _():
            scat_a.extend(run_block(b0, ev_a, cols_a, vals_a, sem_a, sem_sa))

        @pl.when(b1 < NBLK)
        def _():
            scat_b.extend(run_block(b1, ev_b, cols_b, vals_b, sem_b, sem_sb))

        @pl.when(b0 < NBLK)
        def _():
            for cp in scat_a:
                cp.wait()

        @pl.when(b0 + 2 * NW < NBLK)
        def _():
            _stage(ei_ref, b0 + 2 * NW, ev_a, sem_a)

        @pl.when(b1 < NBLK)
        def _():
            for cp in scat_b:
                cp.wait()

        @pl.when(b1 + 2 * NW < NBLK)
        def _():
            _stage(ei_ref, b1 + 2 * NW, ev_b, sem_b)

        return carry

    lax.fori_loop(0, PAIRS, pair, 0)
    plsc.subcore_barrier()

    pltpu.sync_copy(acc.at[pl.ds(s * TSLICE, TSLICE)], bounce)
    pltpu.sync_copy(bounce, out_ref.at[pl.ds(c * NPAD + s * TSLICE, TSLICE)])


_gather_pass = pl.kernel(
    _gather_pass_body,
    out_type=jax.ShapeDtypeStruct((NC * NPAD,), jnp.float32),
    mesh=_MESH,
    scratch_types=[
        pltpu.VMEM((NPAD,), jnp.float32),
        pltpu.VMEM((2 * BLK_ROWS, CH), jnp.int32),
        pltpu.VMEM((BLK,), jnp.int32),
        pltpu.VMEM((BLK,), jnp.float32),
        pltpu.VMEM((2 * BLK_ROWS, CH), jnp.int32),
        pltpu.VMEM((BLK,), jnp.int32),
        pltpu.VMEM((BLK,), jnp.float32),
        pltpu.VMEM((TSLICE,), jnp.float32),
        pltpu.VMEM_SHARED((NPAD,), jnp.float32),
        pltpu.SemaphoreType.DMA,
        pltpu.SemaphoreType.DMA,
        pltpu.SemaphoreType.DMA,
        pltpu.SemaphoreType.DMA,
    ],
    compiler_params=_SC_PARAMS,
)


def _stage_cols(ei_ref, b, cols_v, sem):
    for j in range(BLK_ROWS):
        pltpu.async_copy(ei_ref.at[b * 2 * BLK_ROWS + 2 * j + 1],
                         cols_v.at[pl.ds(j * CH, CH)], sem)


def _wait_cols(ei_ref, b, cols_v, sem):
    for j in range(BLK_ROWS):
        pltpu.make_async_copy(ei_ref.at[b * 2 * BLK_ROWS + 2 * j + 1],
                              cols_v.at[pl.ds(j * CH, CH)], sem).wait()


def _deg_pass_body(ei_ref, zero_ref, out_ref, cols_a, cols_b, ones_v,
                   bounce, acc, sem_a, sem_b, sem_sa, sem_sb):
    c = lax.axis_index("c")
    s = lax.axis_index("s")
    wid = s * NC + c

    for i in range(BLK // L):
        ones_v[pl.ds(i * L, L)] = jnp.ones((L,), jnp.float32)
    pltpu.sync_copy(zero_ref.at[pl.ds(s * TSLICE, TSLICE)],
                    acc.at[pl.ds(s * TSLICE, TSLICE)])
    plsc.subcore_barrier()

    _stage_cols(ei_ref, wid, cols_a, sem_a)
    _stage_cols(ei_ref, wid + NW, cols_b, sem_b)

    def run_block(b, cols_v, sem, sem_s):
        _wait_cols(ei_ref, b, cols_v, sem)
        return [pltpu.async_copy(ones_v, acc.at[cols_v], sem_s, add=True)]

    def pair(k2, carry):
        b0 = wid + NW * (2 * k2)
        b1 = b0 + NW
        scat_a = []
        scat_b = []

        @pl.when(b0 < NBLK)
        def _():
            scat_a.extend(run_block(b0, cols_a, sem_a, sem_sa))

        @pl.when(b1 < NBLK)
        def _():
            scat_b.extend(run_block(b1, cols_b, sem_b, sem_sb))

        @pl.when(b0 < NBLK)
        def _():
            for cp in scat_a:
                cp.wait()

        @pl.when(b0 + 2 * NW < NBLK)
        def _():
            _stage_cols(ei_ref, b0 + 2 * NW, cols_a, sem_a)

        @pl.when(b1 < NBLK)
        def _():
            for cp in scat_b:
                cp.wait()

        @pl.when(b1 + 2 * NW < NBLK)
        def _():
            _stage_cols(ei_ref, b1 + 2 * NW, cols_b, sem_b)

        return carry

    lax.fori_loop(0, PAIRS, pair, 0)
    plsc.subcore_barrier()

    pltpu.sync_copy(acc.at[pl.ds(s * TSLICE, TSLICE)], bounce)
    pltpu.sync_copy(bounce, out_ref.at[pl.ds(c * NPAD + s * TSLICE, TSLICE)])


_deg_pass = pl.kernel(
    _deg_pass_body,
    out_type=jax.ShapeDtypeStruct((NC * NPAD,), jnp.float32),
    mesh=_MESH,
    scratch_types=[
        pltpu.VMEM((BLK,), jnp.int32),
        pltpu.VMEM((BLK,), jnp.int32),
        pltpu.VMEM((BLK,), jnp.float32),
        pltpu.VMEM((TSLICE,), jnp.float32),
        pltpu.VMEM_SHARED((NPAD,), jnp.float32),
        pltpu.SemaphoreType.DMA,
        pltpu.SemaphoreType.DMA,
        pltpu.SemaphoreType.DMA,
        pltpu.SemaphoreType.DMA,
    ],
    compiler_params=_SC_PARAMS,
)


def _ew1_body(d_ref, x_ref, dis_ref, g_ref):
    deg = d_ref[0] + d_ref[1] + 1.0
    dis = lax.rsqrt(deg)
    dis_ref[...] = dis
    g_ref[...] = dis * x_ref[...]


def _ew2_body(p_ref, g_ref, dis_ref, w1_ref, b1_ref, w2_ref, g2_ref):
    dis = dis_ref[...]
    sv = dis * (p_ref[0] + p_ref[1] + g_ref[...])
    t = jnp.zeros_like(sv)
    for k in range(16):
        t = t + jnp.maximum(sv * w1_ref[0, k] + b1_ref[k], 0.0) * w2_ref[k, 0]
    g2_ref[...] = dis * t


def _ew3_body(p_ref, g2_ref, dis_ref, b2_ref, o_ref):
    u = dis_ref[...] * (p_ref[0] + p_ref[1] + g2_ref[...]) + b2_ref[0]
    o_ref[...] = 1.0 / (1.0 + jnp.exp(-u))


_V = functools.partial(pl.BlockSpec, memory_space=pltpu.MemorySpace.VMEM)
_S = functools.partial(pl.BlockSpec, memory_space=pltpu.MemorySpace.SMEM)
_F = jax.ShapeDtypeStruct((ROWS128, 128), jnp.float32)

_ew1 = pl.pallas_call(_ew1_body, out_shape=(_F, _F),
                      in_specs=[_V(), _V()], out_specs=(_V(), _V()))
_ew2 = pl.pallas_call(_ew2_body, out_shape=_F,
                      in_specs=[_V(), _V(), _V(), _S(), _S(), _S()],
                      out_specs=_V())
_ew3 = pl.pallas_call(_ew3_body, out_shape=_F,
                      in_specs=[_V(), _V(), _V(), _S()], out_specs=_V())


def kernel(x, edge_index, W1, b1, W2, b2):
    ei = (edge_index.astype(jnp.int32)
          .reshape(2, N_EDGES // CH, CH)
          .transpose(1, 0, 2)
          .reshape(2 * (N_EDGES // CH), CH))
    xp = jnp.pad(x.astype(jnp.float32), (0, NPAD - N_NODES))
    zeros = jnp.zeros((NPAD,), jnp.float32)

    deg_p = _deg_pass(ei, zeros)
    dis, g = _ew1(deg_p.reshape(NC, ROWS128, 128), xp.reshape(ROWS128, 128))

    s_p = _gather_pass(ei, g.reshape(NPAD), zeros)
    g2 = _ew2(s_p.reshape(NC, ROWS128, 128), g, dis, W1, b1, W2)

    u_p = _gather_pass(ei, g2.reshape(NPAD), zeros)
    out = _ew3(u_p.reshape(NC, ROWS128, 128), g2, dis, b2)

    return out.reshape(NPAD)[:N_NODES].reshape(N_NODES, 1)


# speedup vs baseline: 517.6482x; 1.3164x over previous
import functools

import jax
import jax.numpy as jnp
from jax import lax
from jax.experimental import pallas as pl
from jax.experimental.pallas import tpu as pltpu
from jax.experimental.pallas import tpu_sc as plsc

N_NODES = 100000
N_EDGES = 3200000
NC, NS, L = 2, 16, 16
NW = NC * NS
CH = 128
BLK_ROWS = 8
BLK = BLK_ROWS * CH
NBLK = N_EDGES // BLK
KMAX = -(-NBLK // NW)
PAIRS = (KMAX + 1) // 2
NPAD = 102400
ROWS128 = NPAD // 128
TSLICE = NPAD // NS

_SC_PARAMS = pltpu.CompilerParams(needs_layout_passes=False)
_MESH = plsc.VectorSubcoreMesh(core_axis_name="c", subcore_axis_name="s",
                               num_cores=NC, num_subcores=NS)


def _stage(ei_ref, b, dst, sem):
    return pltpu.async_copy(
        ei_ref.at[pl.ds(b * 2 * BLK_ROWS, 2 * BLK_ROWS)], dst, sem)


def _gather_pass_body(ei_ref, g_ref, zero_ref, out_ref, gtab,
                      ev0, co0, va0, ev1, co1, va1,
                      ev2, co2, va2, ev3, co3, va3,
                      bounce, acc,
                      sg0, sg1, sg2, sg3, ss0, ss1, ss2, ss3):
    c = lax.axis_index("c")
    s = lax.axis_index("s")
    wid = s * NC + c
    evs = (ev0, ev1, ev2, ev3)
    cos = (co0, co1, co2, co3)
    vas = (va0, va1, va2, va3)
    sgs = (sg0, sg1, sg2, sg3)
    sss = (ss0, ss1, ss2, ss3)

    pltpu.sync_copy(g_ref, gtab)
    pltpu.sync_copy(zero_ref.at[pl.ds(s * TSLICE, TSLICE)],
                    acc.at[pl.ds(s * TSLICE, TSLICE)])
    plsc.subcore_barrier()

    _stage(ei_ref, wid, ev0, sg0)
    _stage(ei_ref, wid + NW, ev1, sg1)

    def quad(k4, carry):
        for i in range(4):
            j4 = 4 * k4 + i
            b = wid + NW * j4
            q = (i + 2) % 4

            @pl.when(b < NBLK)
            def _(i=i, b=b):
                ev_v, cols_v, vals_v = evs[i], cos[i], vas[i]
                pltpu.make_async_copy(
                    ei_ref.at[pl.ds(b * 2 * BLK_ROWS, 2 * BLK_ROWS)],
                    ev_v, sgs[i]).wait()
                for j in range(BLK_ROWS):
                    for k in range(CH // L):
                        idx = ev_v[2 * j, pl.ds(k * L, L)]
                        vals_v[pl.ds(j * CH + k * L, L)] = (
                            plsc.load_gather(gtab, [idx]))
                        cols_v[pl.ds(j * CH + k * L, L)] = (
                            ev_v[2 * j + 1, pl.ds(k * L, L)])
                pltpu.async_copy(vals_v, acc.at[cols_v], sss[i], add=True)

            fired_jm2 = b - 2 * NW < NBLK
            if i < 2:
                fired_jm2 = jnp.logical_and(k4 > 0, fired_jm2)

            @pl.when(fired_jm2)
            def _(q=q):
                pltpu.make_async_copy(vas[q], acc.at[cos[q]], sss[q]).wait()

            @pl.when(b + 2 * NW < NBLK)
            def _(q=q, b=b):
                _stage(ei_ref, b + 2 * NW, evs[q], sgs[q])

        return carry

    lax.fori_loop(0, (KMAX + 2 + 3) // 4, quad, 0)
    plsc.subcore_barrier()

    for h in range(2):
        off = s * TSLICE + h * (TSLICE // 2)
        pltpu.sync_copy(acc.at[pl.ds(off, TSLICE // 2)], bounce)
        pltpu.sync_copy(bounce, out_ref.at[pl.ds(c * NPAD + off, TSLICE // 2)])


_gather_pass = pl.kernel(
    _gather_pass_body,
    out_type=jax.ShapeDtypeStruct((NC * NPAD,), jnp.float32),
    mesh=_MESH,
    scratch_types=(
        [pltpu.VMEM((NPAD,), jnp.float32)]
        + [pltpu.VMEM((2 * BLK_ROWS, CH), jnp.int32) if r == 0
           else pltpu.VMEM((BLK,), jnp.int32) if r == 1
           else pltpu.VMEM((BLK,), jnp.float32)
           for _ in range(4) for r in range(3)]
        + [pltpu.VMEM((TSLICE // 2,), jnp.float32),
           pltpu.VMEM_SHARED((NPAD,), jnp.float32)]
        + [pltpu.SemaphoreType.DMA] * 8
    ),
    compiler_params=_SC_PARAMS,
)


def _stage_cols(ei_ref, b, cols_v, sem):
    for j in range(BLK_ROWS):
        pltpu.async_copy(ei_ref.at[b * 2 * BLK_ROWS + 2 * j + 1],
                         cols_v.at[pl.ds(j * CH, CH)], sem)


def _wait_cols(ei_ref, b, cols_v, sem):
    for j in range(BLK_ROWS):
        pltpu.make_async_copy(ei_ref.at[b * 2 * BLK_ROWS + 2 * j + 1],
                              cols_v.at[pl.ds(j * CH, CH)], sem).wait()


def _deg_pass_body(ei_ref, zero_ref, out_ref,
                   co0, co1, co2, co3, ones_v, bounce, acc,
                   sg0, sg1, sg2, sg3, ss0, ss1, ss2, ss3):
    c = lax.axis_index("c")
    s = lax.axis_index("s")
    wid = s * NC + c
    cos = (co0, co1, co2, co3)
    sgs = (sg0, sg1, sg2, sg3)
    sss = (ss0, ss1, ss2, ss3)

    for i in range(BLK // L):
        ones_v[pl.ds(i * L, L)] = jnp.ones((L,), jnp.float32)
    pltpu.sync_copy(zero_ref.at[pl.ds(s * TSLICE, TSLICE)],
                    acc.at[pl.ds(s * TSLICE, TSLICE)])
    plsc.subcore_barrier()

    _stage_cols(ei_ref, wid, co0, sg0)
    _stage_cols(ei_ref, wid + NW, co1, sg1)

    def quad(k4, carry):
        for i in range(4):
            j4 = 4 * k4 + i
            b = wid + NW * j4
            q = (i + 2) % 4

            @pl.when(b < NBLK)
            def _(i=i, b=b):
                _wait_cols(ei_ref, b, cos[i], sgs[i])
                pltpu.async_copy(ones_v, acc.at[cos[i]], sss[i], add=True)

            fired_jm2 = b - 2 * NW < NBLK
            if i < 2:
                fired_jm2 = jnp.logical_and(k4 > 0, fired_jm2)

            @pl.when(fired_jm2)
            def _(q=q):
                pltpu.make_async_copy(ones_v, acc.at[cos[q]], sss[q]).wait()

            @pl.when(b + 2 * NW < NBLK)
            def _(q=q, b=b):
                _stage_cols(ei_ref, b + 2 * NW, cos[q], sgs[q])

        return carry

    lax.fori_loop(0, (KMAX + 2 + 3) // 4, quad, 0)
    plsc.subcore_barrier()

    pltpu.sync_copy(acc.at[pl.ds(s * TSLICE, TSLICE)], bounce)
    pltpu.sync_copy(bounce, out_ref.at[pl.ds(c * NPAD + s * TSLICE, TSLICE)])


_deg_pass = pl.kernel(
    _deg_pass_body,
    out_type=jax.ShapeDtypeStruct((NC * NPAD,), jnp.float32),
    mesh=_MESH,
    scratch_types=(
        [pltpu.VMEM((BLK,), jnp.int32)] * 4
        + [pltpu.VMEM((BLK,), jnp.float32),
           pltpu.VMEM((TSLICE,), jnp.float32),
           pltpu.VMEM_SHARED((NPAD,), jnp.float32)]
        + [pltpu.SemaphoreType.DMA] * 8
    ),
    compiler_params=_SC_PARAMS,
)


def _ew1_body(d_ref, x_ref, dis_ref, g_ref):
    deg = d_ref[0] + d_ref[1] + 1.0
    dis = lax.rsqrt(deg)
    dis_ref[...] = dis
    g_ref[...] = dis * x_ref[...]


def _ew2_body(p_ref, g_ref, dis_ref, w1_ref, b1_ref, w2_ref, g2_ref):
    dis = dis_ref[...]
    sv = dis * (p_ref[0] + p_ref[1] + g_ref[...])
    t = jnp.zeros_like(sv)
    for k in range(16):
        t = t + jnp.maximum(sv * w1_ref[0, k] + b1_ref[k], 0.0) * w2_ref[k, 0]
    g2_ref[...] = dis * t


def _ew3_body(p_ref, g2_ref, dis_ref, b2_ref, o_ref):
    u = dis_ref[...] * (p_ref[0] + p_ref[1] + g2_ref[...]) + b2_ref[0]
    o_ref[...] = 1.0 / (1.0 + jnp.exp(-u))


_V = functools.partial(pl.BlockSpec, memory_space=pltpu.MemorySpace.VMEM)
_S = functools.partial(pl.BlockSpec, memory_space=pltpu.MemorySpace.SMEM)
_F = jax.ShapeDtypeStruct((ROWS128, 128), jnp.float32)

_ew1 = pl.pallas_call(_ew1_body, out_shape=(_F, _F),
                      in_specs=[_V(), _V()], out_specs=(_V(), _V()))
_ew2 = pl.pallas_call(_ew2_body, out_shape=_F,
                      in_specs=[_V(), _V(), _V(), _S(), _S(), _S()],
                      out_specs=_V())
_ew3 = pl.pallas_call(_ew3_body, out_shape=_F,
                      in_specs=[_V(), _V(), _V(), _S()], out_specs=_V())


def kernel(x, edge_index, W1, b1, W2, b2):
    ei = (edge_index.astype(jnp.int32)
          .reshape(2, N_EDGES // CH, CH)
          .transpose(1, 0, 2)
          .reshape(2 * (N_EDGES // CH), CH))
    xp = jnp.pad(x.astype(jnp.float32), (0, NPAD - N_NODES))
    zeros = jnp.zeros((NPAD,), jnp.float32)

    deg_p = _deg_pass(ei, zeros)
    dis, g = _ew1(deg_p.reshape(NC, ROWS128, 128), xp.reshape(ROWS128, 128))

    s_p = _gather_pass(ei, g.reshape(NPAD), zeros)
    g2 = _ew2(s_p.reshape(NC, ROWS128, 128), g, dis, W1, b1, W2)

    u_p = _gather_pass(ei, g2.reshape(NPAD), zeros)
    out = _ew3(u_p.reshape(NC, ROWS128, 128), g2, dis, b2)

    return out.reshape(NPAD)[:N_NODES].reshape(N_NODES, 1)
